# Initial kernel scaffold; baseline (speedup 1.0000x reference)
import functools

import jax
import jax.numpy as jnp
from jax import lax
from jax.experimental import pallas as pl
from jax.experimental.pallas import tpu as pltpu
from jax.experimental.pallas import tpu_sc as plsc

N_NODES = 10000
N_EDGES = 320000
D_IN = 128
D_HID = 128
D_OUT = 64

NC = 2
NS = 16
NW = NC * NS
EPW = N_EDGES // NW
CHUNK = 125
NCHUNK = EPW // CHUNK
ROWS_PER_TILE = N_NODES // NS
DEG_W = 16

_mesh = plsc.VectorSubcoreMesh(core_axis_name="c", subcore_axis_name="s")


def _fill_vmem_2d(ref, nrow, ncol, value):
    v16 = jnp.full((16,), value, jnp.float32)

    def row(i, _):
        for c in range(ncol // 16):
            ref[i, pl.ds(c * 16, 16)] = v16
        return 0

    lax.fori_loop(0, nrow, row, 0)


def _deg_body(dst_hbm, out_hbm, dstv, obuf, acc, sem):
    cid = lax.axis_index("c")
    sid = lax.axis_index("s")
    w = cid * NS + sid

    _fill_vmem_2d(obuf, CHUNK, DEG_W, 0.0)
    for k in range(ROWS_PER_TILE // CHUNK):
        pltpu.sync_copy(obuf, acc.at[pl.ds(sid * ROWS_PER_TILE + k * CHUNK, CHUNK)])
    _fill_vmem_2d(obuf, CHUNK, DEG_W, 1.0)
    plsc.subcore_barrier()

    pltpu.sync_copy(dst_hbm.at[w], dstv)

    def chunk(j, _):
        pltpu.sync_copy(obuf, acc.at[dstv.at[j]], add=True)
        return 0

    lax.fori_loop(0, NCHUNK, chunk, 0)
    plsc.subcore_barrier()

    pltpu.sync_copy(
        acc.at[pl.ds(sid * ROWS_PER_TILE, ROWS_PER_TILE)],
        out_hbm.at[cid, pl.ds(sid * ROWS_PER_TILE, ROWS_PER_TILE)],
    )


_deg_kernel = pl.kernel(
    _deg_body,
    out_type=jax.ShapeDtypeStruct((NC, N_NODES, DEG_W), jnp.float32),
    mesh=_mesh,
    scratch_types=[
        pltpu.VMEM((NCHUNK, CHUNK), jnp.int32),
        pltpu.VMEM((CHUNK, DEG_W), jnp.float32),
        pltpu.VMEM_SHARED((N_NODES, DEG_W), jnp.float32),
        pltpu.SemaphoreType.DMA,
    ],
)


def _agg_body(src_hbm, dst_hbm, u_hbm, out_hbm, srcv, dstv, rows, acc, sem, *, d):
    cid = lax.axis_index("c")
    sid = lax.axis_index("s")
    w = cid * NS + sid

    _fill_vmem_2d(rows, CHUNK, d, 0.0)
    for k in range(ROWS_PER_TILE // CHUNK):
        pltpu.sync_copy(rows, acc.at[pl.ds(sid * ROWS_PER_TILE + k * CHUNK, CHUNK)])
    plsc.subcore_barrier()

    pltpu.sync_copy(src_hbm.at[w], srcv)
    pltpu.sync_copy(dst_hbm.at[w], dstv)

    def chunk(j, _):
        pltpu.async_copy(u_hbm.at[srcv.at[j]], rows, sem).wait()
        pltpu.sync_copy(rows, acc.at[dstv.at[j]], add=True)
        return 0

    lax.fori_loop(0, NCHUNK, chunk, 0)
    plsc.subcore_barrier()

    pltpu.sync_copy(
        acc.at[pl.ds(sid * ROWS_PER_TILE, ROWS_PER_TILE)],
        out_hbm.at[cid, pl.ds(sid * ROWS_PER_TILE, ROWS_PER_TILE)],
    )


def _make_agg(d):
    return pl.kernel(
        functools.partial(_agg_body, d=d),
        out_type=jax.ShapeDtypeStruct((NC, N_NODES, d), jnp.float32),
        mesh=_mesh,
        scratch_types=[
            pltpu.VMEM((NCHUNK, CHUNK), jnp.int32),
            pltpu.VMEM((NCHUNK, CHUNK), jnp.int32),
            pltpu.VMEM((CHUNK, d), jnp.float32),
            pltpu.VMEM_SHARED((N_NODES, d), jnp.float32),
            pltpu.SemaphoreType.DMA,
        ],
    )


_agg128 = _make_agg(D_HID)
_agg64 = _make_agg(D_OUT)


_BR = 1000
_GRID = (N_NODES // _BR,)


def _tc_first_body(x_ref, w_ref, deg_ref, u_ref, dis_ref):
    deg = deg_ref[0, :, 0:1] + deg_ref[1, :, 0:1] + 1.0
    dis = lax.rsqrt(deg)
    h = jnp.dot(x_ref[...], w_ref[...], preferred_element_type=jnp.float32)
    u_ref[...] = h * dis
    dis_ref[...] = jnp.broadcast_to(dis, dis_ref.shape)


_tc_first = pl.pallas_call(
    _tc_first_body,
    grid=_GRID,
    in_specs=[
        pl.BlockSpec((_BR, D_IN), lambda i: (i, 0)),
        pl.BlockSpec((D_IN, D_HID), lambda i: (0, 0)),
        pl.BlockSpec((NC, _BR, DEG_W), lambda i: (0, i, 0)),
    ],
    out_specs=[
        pl.BlockSpec((_BR, D_HID), lambda i: (i, 0)),
        pl.BlockSpec((_BR, D_HID), lambda i: (i, 0)),
    ],
    out_shape=[
        jax.ShapeDtypeStruct((N_NODES, D_HID), jnp.float32),
        jax.ShapeDtypeStruct((N_NODES, D_HID), jnp.float32),
    ],
)


def _tc_mid_body(p_ref, u_ref, dis_ref, b_ref, w_ref, un_ref, *, dout):
    dis = dis_ref[...]
    t = dis * (p_ref[0] + p_ref[1] + u_ref[...]) + b_ref[...]
    t = jnp.maximum(t, 0.0)
    h = jnp.dot(t, w_ref[...], preferred_element_type=jnp.float32)
    un_ref[...] = h * dis[:, :dout]


def _make_tc_mid(dout):
    return pl.pallas_call(
        functools.partial(_tc_mid_body, dout=dout),
        grid=_GRID,
        in_specs=[
            pl.BlockSpec((NC, _BR, D_HID), lambda i: (0, i, 0)),
            pl.BlockSpec((_BR, D_HID), lambda i: (i, 0)),
            pl.BlockSpec((_BR, D_HID), lambda i: (i, 0)),
            pl.BlockSpec((1, D_HID), lambda i: (0, 0)),
            pl.BlockSpec((D_HID, dout), lambda i: (0, 0)),
        ],
        out_specs=pl.BlockSpec((_BR, dout), lambda i: (i, 0)),
        out_shape=jax.ShapeDtypeStruct((N_NODES, dout), jnp.float32),
    )


_tc_mid2 = _make_tc_mid(D_HID)
_tc_mid3 = _make_tc_mid(D_OUT)


def _tc_last_body(p_ref, u_ref, dis_ref, b_ref, out_ref):
    dis = dis_ref[...][:, :D_OUT]
    out_ref[...] = dis * (p_ref[0] + p_ref[1] + u_ref[...]) + b_ref[...]


_tc_last = pl.pallas_call(
    _tc_last_body,
    grid=_GRID,
    in_specs=[
        pl.BlockSpec((NC, _BR, D_OUT), lambda i: (0, i, 0)),
        pl.BlockSpec((_BR, D_OUT), lambda i: (i, 0)),
        pl.BlockSpec((_BR, D_HID), lambda i: (i, 0)),
        pl.BlockSpec((1, D_OUT), lambda i: (0, 0)),
    ],
    out_specs=pl.BlockSpec((_BR, D_OUT), lambda i: (i, 0)),
    out_shape=jax.ShapeDtypeStruct((N_NODES, D_OUT), jnp.float32),
)


def kernel(x, edge_index, W1, b1, W2, b2, W3, b3):
    src = edge_index[0].astype(jnp.int32).reshape(NW, NCHUNK, CHUNK)
    dst = edge_index[1].astype(jnp.int32).reshape(NW, NCHUNK, CHUNK)

    degp = _deg_kernel(dst)
    u1, disB = _tc_first(x, W1, degp)
    p1 = _agg128(src, dst, u1)
    u2 = _tc_mid2(p1, u1, disB, b1.reshape(1, D_HID), W2)
    p2 = _agg128(src, dst, u2)
    u3 = _tc_mid3(p2, u2, disB, b2.reshape(1, D_HID), W3)
    p3 = _agg64(src, dst, u3)
    out = _tc_last(p3, u3, disB, b3.reshape(1, D_OUT))
    return out


# speedup vs baseline: 6.7733x; 6.7733x over previous
import functools

import jax
import jax.numpy as jnp
from jax import lax
from jax.experimental import pallas as pl
from jax.experimental.pallas import tpu as pltpu
from jax.experimental.pallas import tpu_sc as plsc

N_NODES = 10000
N_EDGES = 320000
D_IN = 128
D_HID = 128
D_OUT = 64

NC = 2
NS = 16
NW = NC * NS
N_PAD = 10240
PAD_ROW = N_PAD - 1
CHUNK = 128
NCHUNK = 80
EPW = NCHUNK * CHUNK
E_PAD = NW * EPW
ROWS_PER_TILE = N_PAD // NS
DEG_W = 128

_mesh = plsc.VectorSubcoreMesh(core_axis_name="c", subcore_axis_name="s")


def _fill_vmem_2d(ref, nrow, ncol, value):
    v16 = jnp.full((16,), value, jnp.float32)

    def row(i, _):
        for c in range(ncol // 16):
            ref[i, pl.ds(c * 16, 16)] = v16
        return 0

    lax.fori_loop(0, nrow, row, 0)


def _deg_body(dst_hbm, out_hbm, dstv, obuf, acc, sem):
    cid = lax.axis_index("c")
    sid = lax.axis_index("s")
    w = cid * NS + sid

    _fill_vmem_2d(obuf, CHUNK, DEG_W, 0.0)
    for k in range(ROWS_PER_TILE // CHUNK):
        pltpu.sync_copy(obuf, acc.at[pl.ds(sid * ROWS_PER_TILE + k * CHUNK, CHUNK)])
    _fill_vmem_2d(obuf, CHUNK, DEG_W, 1.0)
    plsc.subcore_barrier()

    pltpu.sync_copy(dst_hbm.at[w], dstv)

    def chunk(j, _):
        pltpu.sync_copy(obuf, acc.at[dstv.at[j]], add=True)
        return 0

    lax.fori_loop(0, NCHUNK, chunk, 0)
    plsc.subcore_barrier()

    pltpu.sync_copy(
        acc.at[pl.ds(sid * ROWS_PER_TILE, ROWS_PER_TILE)],
        out_hbm.at[cid, pl.ds(sid * ROWS_PER_TILE, ROWS_PER_TILE)],
    )


_deg_kernel = pl.kernel(
    _deg_body,
    out_type=jax.ShapeDtypeStruct((NC, N_PAD, DEG_W), jnp.float32),
    mesh=_mesh,
    scratch_types=[
        pltpu.VMEM((NCHUNK, CHUNK), jnp.int32),
        pltpu.VMEM((CHUNK, DEG_W), jnp.float32),
        pltpu.VMEM_SHARED((N_PAD, DEG_W), jnp.float32),
        pltpu.SemaphoreType.DMA,
    ],
)


def _agg_body(src_hbm, dst_hbm, u_hbm, out_hbm, srcv, dstv, rows, acc, sem, *, d):
    cid = lax.axis_index("c")
    sid = lax.axis_index("s")
    w = cid * NS + sid

    _fill_vmem_2d(rows, CHUNK, d, 0.0)
    for k in range(ROWS_PER_TILE // CHUNK):
        pltpu.sync_copy(rows, acc.at[pl.ds(sid * ROWS_PER_TILE + k * CHUNK, CHUNK)])
    plsc.subcore_barrier()

    pltpu.sync_copy(src_hbm.at[w], srcv)
    pltpu.sync_copy(dst_hbm.at[w], dstv)

    def chunk(j, _):
        pltpu.async_copy(u_hbm.at[srcv.at[j]], rows, sem).wait()
        pltpu.sync_copy(rows, acc.at[dstv.at[j]], add=True)
        return 0

    lax.fori_loop(0, NCHUNK, chunk, 0)
    plsc.subcore_barrier()

    pltpu.sync_copy(
        acc.at[pl.ds(sid * ROWS_PER_TILE, ROWS_PER_TILE)],
        out_hbm.at[cid, pl.ds(sid * ROWS_PER_TILE, ROWS_PER_TILE)],
    )


def _make_agg(d):
    return pl.kernel(
        functools.partial(_agg_body, d=d),
        out_type=jax.ShapeDtypeStruct((NC, N_PAD, d), jnp.float32),
        mesh=_mesh,
        scratch_types=[
            pltpu.VMEM((NCHUNK, CHUNK), jnp.int32),
            pltpu.VMEM((NCHUNK, CHUNK), jnp.int32),
            pltpu.VMEM((CHUNK, d), jnp.float32),
            pltpu.VMEM_SHARED((N_PAD, d), jnp.float32),
            pltpu.SemaphoreType.DMA,
        ],
    )


_agg128 = _make_agg(D_HID)


_BR = 1000
_GRID = (N_NODES // _BR,)


def _tc_first_body(x_ref, w_ref, deg_ref, u_ref, dis_ref):
    deg = deg_ref[0, :, 0:1] + deg_ref[1, :, 0:1] + 1.0
    dis = lax.rsqrt(deg)
    h = jnp.dot(x_ref[...], w_ref[...], preferred_element_type=jnp.float32)
    u_ref[...] = h * dis
    dis_ref[...] = jnp.broadcast_to(dis, dis_ref.shape)


_tc_first = pl.pallas_call(
    _tc_first_body,
    grid=_GRID,
    in_specs=[
        pl.BlockSpec((_BR, D_IN), lambda i: (i, 0)),
        pl.BlockSpec((D_IN, D_HID), lambda i: (0, 0)),
        pl.BlockSpec((NC, _BR, DEG_W), lambda i: (0, i, 0)),
    ],
    out_specs=[
        pl.BlockSpec((_BR, D_HID), lambda i: (i, 0)),
        pl.BlockSpec((_BR, D_HID), lambda i: (i, 0)),
    ],
    out_shape=[
        jax.ShapeDtypeStruct((N_NODES, D_HID), jnp.float32),
        jax.ShapeDtypeStruct((N_NODES, D_HID), jnp.float32),
    ],
)


def _tc_mid_body(p_ref, u_ref, dis_ref, b_ref, w_ref, un_ref, *, dout):
    dis = dis_ref[...]
    t = dis * (p_ref[0] + p_ref[1] + u_ref[...]) + b_ref[...]
    t = jnp.maximum(t, 0.0)
    h = jnp.dot(t, w_ref[...], preferred_element_type=jnp.float32)
    un_ref[...] = h * dis[:, :dout]


def _make_tc_mid(dout):
    return pl.pallas_call(
        functools.partial(_tc_mid_body, dout=dout),
        grid=_GRID,
        in_specs=[
            pl.BlockSpec((NC, _BR, D_HID), lambda i: (0, i, 0)),
            pl.BlockSpec((_BR, D_HID), lambda i: (i, 0)),
            pl.BlockSpec((_BR, D_HID), lambda i: (i, 0)),
            pl.BlockSpec((1, D_HID), lambda i: (0, 0)),
            pl.BlockSpec((D_HID, dout), lambda i: (0, 0)),
        ],
        out_specs=pl.BlockSpec((_BR, dout), lambda i: (i, 0)),
        out_shape=jax.ShapeDtypeStruct((N_NODES, dout), jnp.float32),
    )


_tc_mid2 = _make_tc_mid(D_HID)


def _tc_last_body(p_ref, u_ref, dis_ref, b_ref, out_ref):
    dis = dis_ref[...][:, :D_OUT]
    p = p_ref[0, :, :D_OUT] + p_ref[1, :, :D_OUT]
    out_ref[...] = dis * (p + u_ref[...][:, :D_OUT]) + b_ref[...]


_tc_last = pl.pallas_call(
    _tc_last_body,
    grid=_GRID,
    in_specs=[
        pl.BlockSpec((NC, _BR, D_HID), lambda i: (0, i, 0)),
        pl.BlockSpec((_BR, D_HID), lambda i: (i, 0)),
        pl.BlockSpec((_BR, D_HID), lambda i: (i, 0)),
        pl.BlockSpec((1, D_OUT), lambda i: (0, 0)),
    ],
    out_specs=pl.BlockSpec((_BR, D_OUT), lambda i: (i, 0)),
    out_shape=jax.ShapeDtypeStruct((N_NODES, D_OUT), jnp.float32),
)


def kernel(x, edge_index, W1, b1, W2, b2, W3, b3):
    n_fill = E_PAD - N_EDGES
    src = jnp.concatenate(
        [edge_index[0].astype(jnp.int32), jnp.zeros((n_fill,), jnp.int32)]
    ).reshape(NW, NCHUNK, CHUNK)
    dst = jnp.concatenate(
        [edge_index[1].astype(jnp.int32), jnp.full((n_fill,), PAD_ROW, jnp.int32)]
    ).reshape(NW, NCHUNK, CHUNK)

    W3p = jnp.pad(W3, ((0, 0), (0, D_HID - D_OUT)))

    degp = _deg_kernel(dst)
    u1, disB = _tc_first(x, W1, degp)
    p1 = _agg128(src, dst, u1)
    u2 = _tc_mid2(p1, u1, disB, b1.reshape(1, D_HID), W2)
    p2 = _agg128(src, dst, u2)
    u3 = _tc_mid2(p2, u2, disB, b2.reshape(1, D_HID), W3p)
    p3 = _agg128(src, dst, u3)
    out = _tc_last(p3, u3, disB, b3.reshape(1, D_OUT))
    return out


# speedup vs baseline: 7.3028x; 1.0782x over previous
import functools

import jax
import jax.numpy as jnp
from jax import lax
from jax.experimental import pallas as pl
from jax.experimental.pallas import tpu as pltpu
from jax.experimental.pallas import tpu_sc as plsc

N_NODES = 10000
N_EDGES = 320000
D_IN = 128
D_HID = 128
D_OUT = 64

NC = 2
NS = 16
NW = NC * NS
N_PAD = 10240
PAD_ROW = N_PAD - 1
CHUNK = 128
NCHUNK = 80
EPW = NCHUNK * CHUNK
E_PAD = NW * EPW
ROWS_PER_TILE = N_PAD // NS
DEG_W = 128

_mesh = plsc.VectorSubcoreMesh(core_axis_name="c", subcore_axis_name="s")


def _fill_vmem_2d(ref, nrow, ncol, value):
    v16 = jnp.full((16,), value, jnp.float32)

    def row(i, _):
        for c in range(ncol // 16):
            ref[i, pl.ds(c * 16, 16)] = v16
        return 0

    lax.fori_loop(0, nrow, row, 0)


_DEG_WIN = 8


def _deg_body(dst_hbm, out_hbm, dstv, obuf, acc, sem):
    cid = lax.axis_index("c")
    sid = lax.axis_index("s")
    w = cid * NS + sid

    _fill_vmem_2d(obuf, CHUNK, DEG_W, 0.0)
    for k in range(ROWS_PER_TILE // CHUNK):
        pltpu.sync_copy(obuf, acc.at[pl.ds(sid * ROWS_PER_TILE + k * CHUNK, CHUNK)])
    _fill_vmem_2d(obuf, CHUNK, DEG_W, 1.0)
    plsc.subcore_barrier()

    pltpu.sync_copy(dst_hbm.at[w], dstv)

    def wait_one():
        pltpu.make_async_copy(
            out_hbm.at[0, pl.ds(0, CHUNK)], obuf, sem
        ).wait()

    def chunk(j, _):
        pltpu.async_copy(obuf, acc.at[dstv.at[j]], sem, add=True)

        @pl.when(j >= _DEG_WIN)
        def _():
            wait_one()

        return 0

    lax.fori_loop(0, NCHUNK, chunk, 0)
    for _ in range(_DEG_WIN):
        wait_one()
    plsc.subcore_barrier()

    pltpu.sync_copy(
        acc.at[pl.ds(sid * ROWS_PER_TILE, ROWS_PER_TILE)],
        out_hbm.at[cid, pl.ds(sid * ROWS_PER_TILE, ROWS_PER_TILE)],
    )


_deg_kernel = pl.kernel(
    _deg_body,
    out_type=jax.ShapeDtypeStruct((NC, N_PAD, DEG_W), jnp.float32),
    mesh=_mesh,
    scratch_types=[
        pltpu.VMEM((NCHUNK, CHUNK), jnp.int32),
        pltpu.VMEM((CHUNK, DEG_W), jnp.float32),
        pltpu.VMEM_SHARED((N_PAD, DEG_W), jnp.float32),
        pltpu.SemaphoreType.DMA,
    ],
)


HC = 40


def _agg_body(src_hbm, dst_hbm, u_hbm, out_hbm, srcv, dstv, rows, acc, gsem, ssem, *, d):
    cid = lax.axis_index("c")
    sid = lax.axis_index("s")
    w = cid * NS + sid

    def wait_dma(buf, sem):
        pltpu.make_async_copy(u_hbm.at[pl.ds(0, CHUNK)], rows.at[buf], sem).wait()

    def gather(local_j, buf):
        pltpu.async_copy(u_hbm.at[srcv.at[local_j]], rows.at[buf], gsem.at[buf])

    def scatter(local_j, buf):
        pltpu.async_copy(rows.at[buf], acc.at[dstv.at[local_j]], ssem.at[buf], add=True)

    zeros16 = jnp.zeros((16,), jnp.float32)

    def zrow(i, _):
        for c in range(d // 16):
            rows[1, i, pl.ds(c * 16, 16)] = zeros16
        return 0

    lax.fori_loop(0, CHUNK, zrow, 0)
    for k in range(ROWS_PER_TILE // CHUNK):
        pltpu.sync_copy(
            rows.at[1], acc.at[pl.ds(sid * ROWS_PER_TILE + k * CHUNK, CHUNK)]
        )
    plsc.subcore_barrier()

    for base in (0, HC):
        pltpu.sync_copy(src_hbm.at[w, pl.ds(base, HC)], srcv)
        pltpu.sync_copy(dst_hbm.at[w, pl.ds(base, HC)], dstv)

        gather(0, 0)
        wait_dma(0, gsem.at[0])
        gather(1, 1)
        scatter(0, 0)

        def pair(g, _):
            j1 = 2 * g + 1
            wait_dma(1, gsem.at[1])
            wait_dma(0, ssem.at[0])
            gather(j1 + 1, 0)
            scatter(j1, 1)
            j2 = 2 * g + 2
            wait_dma(0, gsem.at[0])
            wait_dma(1, ssem.at[1])
            gather(j2 + 1, 1)
            scatter(j2, 0)
            return 0

        lax.fori_loop(0, HC // 2 - 1, pair, 0)

        wait_dma(1, gsem.at[1])
        wait_dma(0, ssem.at[0])
        scatter(HC - 1, 1)
        wait_dma(1, ssem.at[1])

    plsc.subcore_barrier()
    pltpu.sync_copy(
        acc.at[pl.ds(sid * ROWS_PER_TILE, ROWS_PER_TILE)],
        out_hbm.at[cid, pl.ds(sid * ROWS_PER_TILE, ROWS_PER_TILE)],
    )


def _make_agg(d):
    return pl.kernel(
        functools.partial(_agg_body, d=d),
        out_type=jax.ShapeDtypeStruct((NC, N_PAD, d), jnp.float32),
        mesh=_mesh,
        scratch_types=[
            pltpu.VMEM((HC, CHUNK), jnp.int32),
            pltpu.VMEM((HC, CHUNK), jnp.int32),
            pltpu.VMEM((2, CHUNK, d), jnp.float32),
            pltpu.VMEM_SHARED((N_PAD, d), jnp.float32),
            pltpu.SemaphoreType.DMA((2,)),
            pltpu.SemaphoreType.DMA((2,)),
        ],
    )


_agg128 = _make_agg(D_HID)


_BR = 1000
_GRID = (N_NODES // _BR,)


def _tc_first_body(x_ref, w_ref, deg_ref, u_ref, dis_ref):
    deg = deg_ref[0, :, 0:1] + deg_ref[1, :, 0:1] + 1.0
    dis = lax.rsqrt(deg)
    h = jnp.dot(x_ref[...], w_ref[...], preferred_element_type=jnp.float32)
    u_ref[...] = h * dis
    dis_ref[...] = jnp.broadcast_to(dis, dis_ref.shape)


_tc_first = pl.pallas_call(
    _tc_first_body,
    grid=_GRID,
    in_specs=[
        pl.BlockSpec((_BR, D_IN), lambda i: (i, 0)),
        pl.BlockSpec((D_IN, D_HID), lambda i: (0, 0)),
        pl.BlockSpec((NC, _BR, DEG_W), lambda i: (0, i, 0)),
    ],
    out_specs=[
        pl.BlockSpec((_BR, D_HID), lambda i: (i, 0)),
        pl.BlockSpec((_BR, D_HID), lambda i: (i, 0)),
    ],
    out_shape=[
        jax.ShapeDtypeStruct((N_NODES, D_HID), jnp.float32),
        jax.ShapeDtypeStruct((N_NODES, D_HID), jnp.float32),
    ],
)


def _tc_mid_body(p_ref, u_ref, dis_ref, b_ref, w_ref, un_ref, *, dout):
    dis = dis_ref[...]
    t = dis * (p_ref[0] + p_ref[1] + u_ref[...]) + b_ref[...]
    t = jnp.maximum(t, 0.0)
    h = jnp.dot(t, w_ref[...], preferred_element_type=jnp.float32)
    un_ref[...] = h * dis[:, :dout]


def _make_tc_mid(dout):
    return pl.pallas_call(
        functools.partial(_tc_mid_body, dout=dout),
        grid=_GRID,
        in_specs=[
            pl.BlockSpec((NC, _BR, D_HID), lambda i: (0, i, 0)),
            pl.BlockSpec((_BR, D_HID), lambda i: (i, 0)),
            pl.BlockSpec((_BR, D_HID), lambda i: (i, 0)),
            pl.BlockSpec((1, D_HID), lambda i: (0, 0)),
            pl.BlockSpec((D_HID, dout), lambda i: (0, 0)),
        ],
        out_specs=pl.BlockSpec((_BR, dout), lambda i: (i, 0)),
        out_shape=jax.ShapeDtypeStruct((N_NODES, dout), jnp.float32),
    )


_tc_mid2 = _make_tc_mid(D_HID)


def _tc_last_body(p_ref, u_ref, dis_ref, b_ref, out_ref):
    dis = dis_ref[...][:, :D_OUT]
    p = p_ref[0, :, :D_OUT] + p_ref[1, :, :D_OUT]
    out_ref[...] = dis * (p + u_ref[...][:, :D_OUT]) + b_ref[...]


_tc_last = pl.pallas_call(
    _tc_last_body,
    grid=_GRID,
    in_specs=[
        pl.BlockSpec((NC, _BR, D_HID), lambda i: (0, i, 0)),
        pl.BlockSpec((_BR, D_HID), lambda i: (i, 0)),
        pl.BlockSpec((_BR, D_HID), lambda i: (i, 0)),
        pl.BlockSpec((1, D_OUT), lambda i: (0, 0)),
    ],
    out_specs=pl.BlockSpec((_BR, D_OUT), lambda i: (i, 0)),
    out_shape=jax.ShapeDtypeStruct((N_NODES, D_OUT), jnp.float32),
)


def kernel(x, edge_index, W1, b1, W2, b2, W3, b3):
    n_fill = E_PAD - N_EDGES
    src = jnp.concatenate(
        [edge_index[0].astype(jnp.int32), jnp.zeros((n_fill,), jnp.int32)]
    ).reshape(NW, NCHUNK, CHUNK)
    dst = jnp.concatenate(
        [edge_index[1].astype(jnp.int32), jnp.full((n_fill,), PAD_ROW, jnp.int32)]
    ).reshape(NW, NCHUNK, CHUNK)

    W3p = jnp.pad(W3, ((0, 0), (0, D_HID - D_OUT)))

    degp = _deg_kernel(dst)
    u1, disB = _tc_first(x, W1, degp)
    p1 = _agg128(src, dst, u1)
    u2 = _tc_mid2(p1, u1, disB, b1.reshape(1, D_HID), W2)
    p2 = _agg128(src, dst, u2)
    u3 = _tc_mid2(p2, u2, disB, b2.reshape(1, D_HID), W3p)
    p3 = _agg128(src, dst, u3)
    out = _tc_last(p3, u3, disB, b3.reshape(1, D_OUT))
    return out


# speedup vs baseline: 8.4267x; 1.1539x over previous
import functools

import jax
import jax.numpy as jnp
from jax import lax
from jax.experimental import pallas as pl
from jax.experimental.pallas import tpu as pltpu
from jax.experimental.pallas import tpu_sc as plsc

N_NODES = 10000
N_EDGES = 320000
D_IN = 128
D_HID = 128
D_OUT = 64

NC = 2
NS = 16
NW = NC * NS
N_PAD = 10112
PAD_ROW = N_PAD - 1
CHUNK = 128
K0 = 128
K1 = 32
HC = K0 // 2
NCHK = NS * (K0 + K1)
NCHK_PAD = NCHK + HC
ROWS_PER_TILE = N_PAD // NS
DEG_CH = NCHK // NW
DEG_W = 128

_mesh = plsc.VectorSubcoreMesh(core_axis_name="c", subcore_axis_name="s")


def _fill_vmem_2d(ref, nrow, ncol, value):
    v16 = jnp.full((16,), value, jnp.float32)

    def row(i, _):
        for c in range(ncol // 16):
            ref[i, pl.ds(c * 16, 16)] = v16
        return 0

    lax.fori_loop(0, nrow, row, 0)


def _zero_acc_slice(rows, acc, sid, d):
    zeros16 = jnp.zeros((16,), jnp.float32)

    def zrow(i, _):
        for c in range(d // 16):
            rows[1, i, pl.ds(c * 16, 16)] = zeros16
        return 0

    lax.fori_loop(0, CHUNK, zrow, 0)
    base = sid * ROWS_PER_TILE
    for k in range(4):
        pltpu.sync_copy(rows.at[1], acc.at[pl.ds(base + k * CHUNK, CHUNK)])
    pltpu.sync_copy(
        rows.at[1, pl.ds(0, ROWS_PER_TILE - 4 * CHUNK)],
        acc.at[pl.ds(base + 4 * CHUNK, ROWS_PER_TILE - 4 * CHUNK)],
    )


_DEG_WIN = 8


def _deg_body(dst_hbm, out_hbm, dstv, obuf, acc, sem):
    cid = lax.axis_index("c")
    sid = lax.axis_index("s")
    w = cid * NS + sid

    _fill_vmem_2d(obuf, CHUNK, DEG_W, 0.0)
    base = sid * ROWS_PER_TILE
    for k in range(4):
        pltpu.sync_copy(obuf, acc.at[pl.ds(base + k * CHUNK, CHUNK)])
    pltpu.sync_copy(
        obuf.at[pl.ds(0, ROWS_PER_TILE - 4 * CHUNK)],
        acc.at[pl.ds(base + 4 * CHUNK, ROWS_PER_TILE - 4 * CHUNK)],
    )
    _fill_vmem_2d(obuf, CHUNK, DEG_W, 1.0)
    plsc.subcore_barrier()

    pltpu.sync_copy(dst_hbm.at[pl.ds(w * DEG_CH, DEG_CH)], dstv)

    def wait_one():
        pltpu.make_async_copy(out_hbm.at[0, pl.ds(0, CHUNK)], obuf, sem).wait()

    def chunk(j, _):
        pltpu.async_copy(obuf, acc.at[dstv.at[j]], sem, add=True)

        @pl.when(j >= _DEG_WIN)
        def _():
            wait_one()

        return 0

    lax.fori_loop(0, DEG_CH, chunk, 0)
    for _ in range(_DEG_WIN):
        wait_one()
    plsc.subcore_barrier()

    pltpu.sync_copy(
        acc.at[pl.ds(base, ROWS_PER_TILE)],
        out_hbm.at[cid, pl.ds(base, ROWS_PER_TILE)],
    )


_deg_kernel = pl.kernel(
    _deg_body,
    out_type=jax.ShapeDtypeStruct((NC, N_PAD, DEG_W), jnp.float32),
    mesh=_mesh,
    scratch_types=[
        pltpu.VMEM((DEG_CH, CHUNK), jnp.int32),
        pltpu.VMEM((CHUNK, DEG_W), jnp.float32),
        pltpu.VMEM_SHARED((N_PAD, DEG_W), jnp.float32),
        pltpu.SemaphoreType.DMA,
    ],
)


def _agg_body(src_hbm, dst_hbm, u_hbm, out_hbm, srcv, dstv, rows, acc, gsem, ssem, *, d):
    cid = lax.axis_index("c")
    sid = lax.axis_index("s")

    start = jnp.where(cid == 0, K0 * sid, NS * K0 + K1 * sid)
    nproc = jnp.where(cid == 0, HC, K1 // 2)
    npairs = nproc // 2 - 1

    def wait_dma(buf, sem):
        pltpu.make_async_copy(u_hbm.at[pl.ds(0, CHUNK)], rows.at[buf], sem).wait()

    def gather(local_j, buf):
        pltpu.async_copy(u_hbm.at[srcv.at[local_j]], rows.at[buf], gsem.at[buf])

    def scatter(local_j, buf):
        pltpu.async_copy(rows.at[buf], acc.at[dstv.at[local_j]], ssem.at[buf], add=True)

    _zero_acc_slice(rows, acc, sid, d)
    plsc.subcore_barrier()

    for h in range(2):
        cbase = start + h * nproc
        pltpu.sync_copy(src_hbm.at[pl.ds(cbase, HC)], srcv)
        pltpu.sync_copy(dst_hbm.at[pl.ds(cbase, HC)], dstv)

        gather(0, 0)
        wait_dma(0, gsem.at[0])
        gather(1, 1)
        scatter(0, 0)

        def pair(g, _):
            j1 = 2 * g + 1
            wait_dma(1, gsem.at[1])
            wait_dma(0, ssem.at[0])
            gather(j1 + 1, 0)
            scatter(j1, 1)
            j2 = 2 * g + 2
            wait_dma(0, gsem.at[0])
            wait_dma(1, ssem.at[1])
            gather(j2 + 1, 1)
            scatter(j2, 0)
            return 0

        lax.fori_loop(0, npairs, pair, 0)

        wait_dma(1, gsem.at[1])
        wait_dma(0, ssem.at[0])
        scatter(nproc - 1, 1)
        wait_dma(1, ssem.at[1])

    plsc.subcore_barrier()
    pltpu.sync_copy(
        acc.at[pl.ds(sid * ROWS_PER_TILE, ROWS_PER_TILE)],
        out_hbm.at[cid, pl.ds(sid * ROWS_PER_TILE, ROWS_PER_TILE)],
    )


def _make_agg(d):
    return pl.kernel(
        functools.partial(_agg_body, d=d),
        out_type=jax.ShapeDtypeStruct((NC, N_PAD, d), jnp.float32),
        mesh=_mesh,
        scratch_types=[
            pltpu.VMEM((HC, CHUNK), jnp.int32),
            pltpu.VMEM((HC, CHUNK), jnp.int32),
            pltpu.VMEM((2, CHUNK, d), jnp.float32),
            pltpu.VMEM_SHARED((N_PAD, d), jnp.float32),
            pltpu.SemaphoreType.DMA((2,)),
            pltpu.SemaphoreType.DMA((2,)),
        ],
    )


_agg128 = _make_agg(D_HID)


_BR = 1000
_GRID = (N_NODES // _BR,)


def _tc_first_body(x_ref, w_ref, deg_ref, u_ref, dis_ref):
    deg = deg_ref[0, :, 0:1] + deg_ref[1, :, 0:1] + 1.0
    dis = lax.rsqrt(deg)
    h = jnp.dot(x_ref[...], w_ref[...], preferred_element_type=jnp.float32)
    u_ref[...] = h * dis
    dis_ref[...] = jnp.broadcast_to(dis, dis_ref.shape)


_tc_first = pl.pallas_call(
    _tc_first_body,
    grid=_GRID,
    in_specs=[
        pl.BlockSpec((_BR, D_IN), lambda i: (i, 0)),
        pl.BlockSpec((D_IN, D_HID), lambda i: (0, 0)),
        pl.BlockSpec((NC, _BR, DEG_W), lambda i: (0, i, 0)),
    ],
    out_specs=[
        pl.BlockSpec((_BR, D_HID), lambda i: (i, 0)),
        pl.BlockSpec((_BR, D_HID), lambda i: (i, 0)),
    ],
    out_shape=[
        jax.ShapeDtypeStruct((N_NODES, D_HID), jnp.float32),
        jax.ShapeDtypeStruct((N_NODES, D_HID), jnp.float32),
    ],
)


def _tc_mid_body(p_ref, u_ref, dis_ref, b_ref, w_ref, un_ref, *, dout):
    dis = dis_ref[...]
    t = dis * (p_ref[0] + p_ref[1] + u_ref[...]) + b_ref[...]
    t = jnp.maximum(t, 0.0)
    h = jnp.dot(t, w_ref[...], preferred_element_type=jnp.float32)
    un_ref[...] = h * dis[:, :dout]


def _make_tc_mid(dout):
    return pl.pallas_call(
        functools.partial(_tc_mid_body, dout=dout),
        grid=_GRID,
        in_specs=[
            pl.BlockSpec((NC, _BR, D_HID), lambda i: (0, i, 0)),
            pl.BlockSpec((_BR, D_HID), lambda i: (i, 0)),
            pl.BlockSpec((_BR, D_HID), lambda i: (i, 0)),
            pl.BlockSpec((1, D_HID), lambda i: (0, 0)),
            pl.BlockSpec((D_HID, dout), lambda i: (0, 0)),
        ],
        out_specs=pl.BlockSpec((_BR, dout), lambda i: (i, 0)),
        out_shape=jax.ShapeDtypeStruct((N_NODES, dout), jnp.float32),
    )


_tc_mid2 = _make_tc_mid(D_HID)


def _tc_last_body(p_ref, u_ref, dis_ref, b_ref, out_ref):
    dis = dis_ref[...][:, :D_OUT]
    p = p_ref[0, :, :D_OUT] + p_ref[1, :, :D_OUT]
    out_ref[...] = dis * (p + u_ref[...][:, :D_OUT]) + b_ref[...]


_tc_last = pl.pallas_call(
    _tc_last_body,
    grid=_GRID,
    in_specs=[
        pl.BlockSpec((NC, _BR, D_HID), lambda i: (0, i, 0)),
        pl.BlockSpec((_BR, D_HID), lambda i: (i, 0)),
        pl.BlockSpec((_BR, D_HID), lambda i: (i, 0)),
        pl.BlockSpec((1, D_OUT), lambda i: (0, 0)),
    ],
    out_specs=pl.BlockSpec((_BR, D_OUT), lambda i: (i, 0)),
    out_shape=jax.ShapeDtypeStruct((N_NODES, D_OUT), jnp.float32),
)


def kernel(x, edge_index, W1, b1, W2, b2, W3, b3):
    n_fill = NCHK_PAD * CHUNK - N_EDGES
    src = jnp.concatenate(
        [edge_index[0].astype(jnp.int32), jnp.zeros((n_fill,), jnp.int32)]
    ).reshape(NCHK_PAD, CHUNK)
    dst = jnp.concatenate(
        [edge_index[1].astype(jnp.int32), jnp.full((n_fill,), PAD_ROW, jnp.int32)]
    ).reshape(NCHK_PAD, CHUNK)

    W3p = jnp.pad(W3, ((0, 0), (0, D_HID - D_OUT)))

    degp = _deg_kernel(dst)
    u1, disB = _tc_first(x, W1, degp)
    p1 = _agg128(src, dst, u1)
    u2 = _tc_mid2(p1, u1, disB, b1.reshape(1, D_HID), W2)
    p2 = _agg128(src, dst, u2)
    u3 = _tc_mid2(p2, u2, disB, b2.reshape(1, D_HID), W3p)
    p3 = _agg128(src, dst, u3)
    out = _tc_last(p3, u3, disB, b3.reshape(1, D_OUT))
    return out
